# baseline (speedup 1.0000x reference)
import functools

import jax
import jax.numpy as jnp
from jax import lax
from jax.experimental import pallas as pl
from jax.experimental.pallas import tpu as pltpu
from jax.experimental.pallas import tpu_sc as plsc

IMAGE_FEAT_DIM = 64
FUSED_IMAGE_DIM = 16
TIME_COL = 6

_NC = 2
_NS = 16
_NW = _NC * _NS
_GB = 64


def _tmin_body(times_ref, tmin_ref):
    b = pl.program_id(0)
    tmin_ref[b, 0, 0] = jnp.min(times_ref[0, 0, :])


def _prep_body(pts_t_ref, tmin_ref, t_ref, p_ref, hw_ref,
               idx_ref, w_ref, cur_ref, m_ref, *, hf, wf):
    b = pl.program_id(0)
    x = pts_t_ref[0, 0, :]
    y = pts_t_ref[0, 1, :]
    z = pts_t_ref[0, 2, :]
    tm = pts_t_ref[0, TIME_COL, :]

    ct = tmin_ref[b, 0, 0]
    cur = jnp.abs(tm - ct) <= 1e-4

    def trow(i):
        return (x * t_ref[i, 0] + y * t_ref[i, 1] + z * t_ref[i, 2]
                + t_ref[i, 3])

    cp0 = trow(0)
    cp1 = trow(1)
    cp2 = trow(2)
    depth = cp2
    cp3 = x * t_ref[3, 0] + y * t_ref[3, 1] + z * t_ref[3, 2] + t_ref[3, 3]

    def prow(i):
        return (p_ref[i, 0] * cp0 + p_ref[i, 1] * cp1 + p_ref[i, 2] * cp2
                + p_ref[i, 3] * cp3)

    uvw0 = prow(0)
    uvw1 = prow(1)
    uvw2 = prow(2)
    den = jnp.maximum(uvw2, 1e-5)
    u = uvw0 / den
    v = uvw1 / den

    img_h = hw_ref[0]
    img_w = hw_ref[1]
    valid = ((depth > 0.0)
             & (u >= 0.0) & (u <= jnp.maximum(img_w - 1.0, 0.0))
             & (v >= 0.0) & (v <= jnp.maximum(img_h - 1.0, 0.0)))
    gx = 2.0 * (u / jnp.maximum(img_w - 1.0, 1.0)) - 1.0
    gy = 2.0 * (v / jnp.maximum(img_h - 1.0, 1.0)) - 1.0
    xs = (gx + 1.0) * 0.5 * (wf - 1)
    ys = (gy + 1.0) * 0.5 * (hf - 1)
    x0 = jnp.floor(xs)
    y0 = jnp.floor(ys)
    fx = xs - x0
    fy = ys - y0

    m = (valid & cur).astype(jnp.float32)
    taps = ((x0, y0, (1.0 - fx) * (1.0 - fy)),
            (x0 + 1.0, y0, fx * (1.0 - fy)),
            (x0, y0 + 1.0, (1.0 - fx) * fy),
            (x0 + 1.0, y0 + 1.0, fx * fy))
    for ti, (xf, yf, wgt) in enumerate(taps):
        inr = ((xf >= 0.0) & (xf <= wf - 1.0)
               & (yf >= 0.0) & (yf <= hf - 1.0))
        xi = jnp.clip(xf, 0.0, wf - 1.0).astype(jnp.int32)
        yi = jnp.clip(yf, 0.0, hf - 1.0).astype(jnp.int32)
        idx_ref[0, ti, :] = (b * hf + yi) * wf + xi
        w_ref[0, ti, :] = wgt * inr.astype(jnp.float32) * m
    cur_ref[0, 0, :] = cur.astype(jnp.float32)
    m_ref[0, 0, :] = m


def _sc_gather_body(table_ref, idx_ref, w_ref, out_ref,
                    idx_v0, idx_v1, w_v0, w_v1, rows_v0, rows_v1,
                    out_v0, out_v1, sem_s0, sem_s1, sem_g0, sem_g1,
                    sem_o0, sem_o1, *, n_per_batch, nb):
    cid = lax.axis_index("c")
    sid = lax.axis_index("s")
    wid = sid * _NC + cid
    wpb = _NW // nb
    chunk = n_per_batch // wpb
    nblk = chunk // _GB
    bb = wid // wpb
    base = (wid % wpb) * chunk

    bufs = ((idx_v0, w_v0, rows_v0, out_v0, sem_s0, sem_g0, sem_o0),
            (idx_v1, w_v1, rows_v1, out_v1, sem_s1, sem_g1, sem_o1))

    def stage_start(j, k):
        idx_v, w_v, _, _, sem_s, _, _ = bufs[k]
        start = base + j * _GB
        for t in range(4):
            pltpu.async_copy(idx_ref.at[bb, t, pl.ds(start, _GB)],
                             idx_v.at[t], sem_s)
            pltpu.async_copy(w_ref.at[bb, t, pl.ds(start, _GB)],
                             w_v.at[t], sem_s)

    def stage_wait(k):
        idx_v, w_v, _, _, sem_s, _, _ = bufs[k]
        for t in range(4):
            pltpu.make_async_copy(idx_ref.at[bb, t, pl.ds(base, _GB)],
                                  idx_v.at[t], sem_s).wait()
            pltpu.make_async_copy(w_ref.at[bb, t, pl.ds(base, _GB)],
                                  w_v.at[t], sem_s).wait()

    def gather_start(k):
        idx_v, _, rows_v, _, _, sem_g, _ = bufs[k]
        for t in range(4):
            pltpu.async_copy(table_ref.at[idx_v.at[t]], rows_v.at[t], sem_g)

    def gather_wait(k):
        idx_v, _, rows_v, _, _, sem_g, _ = bufs[k]
        for t in range(4):
            pltpu.make_async_copy(table_ref.at[idx_v.at[t]], rows_v.at[t],
                                  sem_g).wait()

    def store_start(j, k):
        _, _, _, out_v, _, _, sem_o = bufs[k]
        start = base + j * _GB
        pltpu.async_copy(out_v, out_ref.at[bb, pl.ds(start, _GB)], sem_o)

    def store_wait(k):
        _, _, _, out_v, _, _, sem_o = bufs[k]
        pltpu.make_async_copy(out_v, out_ref.at[bb, pl.ds(base, _GB)],
                              sem_o).wait()

    def combine(k):
        _, w_v, rows_v, out_v, _, _, _ = bufs[k]

        def grp_body(g, c2):
            g16 = g * 16
            wvec = [w_v[t, pl.ds(g16, 16)] for t in range(4)]
            for kk in range(16):
                p = g16 + kk
                ws = [wvec[t][kk] for t in range(4)]
                for c in range(IMAGE_FEAT_DIM // 16):
                    sl = pl.ds(c * 16, 16)
                    acc = (rows_v[0, p, sl] * ws[0]
                           + rows_v[1, p, sl] * ws[1]
                           + rows_v[2, p, sl] * ws[2]
                           + rows_v[3, p, sl] * ws[3])
                    out_v[p, sl] = acc
            return c2

        lax.fori_loop(0, _GB // 16, grp_body, 0, unroll=False)

    stage_start(0, 0)
    stage_start(1, 1)
    stage_wait(0)
    gather_start(0)

    def body(i, carry):
        for k in range(2):
            j = 2 * i + k
            nk = 1 - k

            @pl.when(j + 1 < nblk)
            def _next():
                stage_wait(nk)
                gather_start(nk)

            gather_wait(k)

            @pl.when(j >= 2)
            def _drain():
                store_wait(k)

            combine(k)
            store_start(j, k)

            @pl.when(j + 2 < nblk)
            def _restage():
                stage_start(j + 2, k)
        return carry

    lax.fori_loop(0, nblk // 2, body, 0, unroll=False)
    store_wait(0)
    store_wait(1)


def _mlp_body(pts_ref, samp_ref, cur_ref, m_ref, w1t_ref, b1_ref, w2t_ref,
              b2_ref, out_ref):
    s = samp_ref[0]
    h = jnp.dot(s, w1t_ref[...], preferred_element_type=jnp.float32)
    h = jnp.maximum(h + b1_ref[...], 0.0)
    f = jnp.dot(h, w2t_ref[...], preferred_element_type=jnp.float32)
    mcol = m_ref[0, 0, :][:, None]
    f = (f + b2_ref[...]) * mcol
    out_ref[0, :, 0:8] = pts_ref[0]
    out_ref[0, :, 8:9] = cur_ref[0, 0, :][:, None]
    out_ref[0, :, 9:10] = mcol
    out_ref[0, :, 10:26] = f


def kernel(pts, img_feats, t_camera_radar, camera_projection, img_shape,
           W1, b1, W2, b2):
    B, N, D = pts.shape
    C, Hf, Wf = img_feats.shape[1], img_feats.shape[2], img_feats.shape[3]
    f32 = jnp.float32

    pts_t = jnp.transpose(pts, (0, 2, 1))
    times = pts_t[:, TIME_COL:TIME_COL + 1, :]
    table = jnp.transpose(img_feats, (0, 2, 3, 1)).reshape(B * Hf * Wf, C)
    table = jnp.concatenate([table, jnp.zeros_like(table)], axis=1)
    hw_f = img_shape.astype(f32)

    tmin = pl.pallas_call(
        _tmin_body,
        grid=(B,),
        in_specs=[pl.BlockSpec((1, 1, N), lambda b: (b, 0, 0))],
        out_specs=pl.BlockSpec(memory_space=pltpu.SMEM),
        out_shape=jax.ShapeDtypeStruct((B, 1, 1), f32),
    )(times)

    BLK = 2048
    NB = N // BLK
    smem = functools.partial(pl.BlockSpec, memory_space=pltpu.SMEM)
    idx4, w4, cur, m = pl.pallas_call(
        functools.partial(_prep_body, hf=Hf, wf=Wf),
        grid=(B, NB),
        in_specs=[
            pl.BlockSpec((1, 8, BLK), lambda b, i: (b, 0, i)),
            smem(),
            smem(),
            smem(),
            smem(),
        ],
        out_specs=[
            pl.BlockSpec((1, 4, BLK), lambda b, i: (b, 0, i)),
            pl.BlockSpec((1, 4, BLK), lambda b, i: (b, 0, i)),
            pl.BlockSpec((1, 1, BLK), lambda b, i: (b, 0, i)),
            pl.BlockSpec((1, 1, BLK), lambda b, i: (b, 0, i)),
        ],
        out_shape=[
            jax.ShapeDtypeStruct((B, 4, N), jnp.int32),
            jax.ShapeDtypeStruct((B, 4, N), f32),
            jax.ShapeDtypeStruct((B, 1, N), f32),
            jax.ShapeDtypeStruct((B, 1, N), f32),
        ],
    )(pts_t, tmin, t_camera_radar, camera_projection, hw_f)

    mesh = plsc.VectorSubcoreMesh(core_axis_name="c", subcore_axis_name="s",
                                  num_cores=_NC, num_subcores=_NS)
    sampled = pl.kernel(
        functools.partial(_sc_gather_body, n_per_batch=N, nb=B),
        out_type=jax.ShapeDtypeStruct((B, N, C), f32),
        mesh=mesh,
        scratch_types=[
            pltpu.VMEM((4, _GB), jnp.int32),
            pltpu.VMEM((4, _GB), jnp.int32),
            pltpu.VMEM((4, _GB), f32),
            pltpu.VMEM((4, _GB), f32),
            pltpu.VMEM((4, _GB, 2 * C), f32),
            pltpu.VMEM((4, _GB, 2 * C), f32),
            pltpu.VMEM((_GB, C), f32),
            pltpu.VMEM((_GB, C), f32),
            pltpu.SemaphoreType.DMA,
            pltpu.SemaphoreType.DMA,
            pltpu.SemaphoreType.DMA,
            pltpu.SemaphoreType.DMA,
            pltpu.SemaphoreType.DMA,
            pltpu.SemaphoreType.DMA,
        ],
    )(table, idx4, w4)

    BLK2 = 2048
    out = pl.pallas_call(
        _mlp_body,
        grid=(B, N // BLK2),
        in_specs=[
            pl.BlockSpec((1, BLK2, 8), lambda b, i: (b, i, 0)),
            pl.BlockSpec((1, BLK2, C), lambda b, i: (b, i, 0)),
            pl.BlockSpec((1, 1, BLK2), lambda b, i: (b, 0, i)),
            pl.BlockSpec((1, 1, BLK2), lambda b, i: (b, 0, i)),
            pl.BlockSpec((C, FUSED_IMAGE_DIM), lambda b, i: (0, 0)),
            pl.BlockSpec((1, FUSED_IMAGE_DIM), lambda b, i: (0, 0)),
            pl.BlockSpec((FUSED_IMAGE_DIM, FUSED_IMAGE_DIM),
                         lambda b, i: (0, 0)),
            pl.BlockSpec((1, FUSED_IMAGE_DIM), lambda b, i: (0, 0)),
        ],
        out_specs=pl.BlockSpec((1, BLK2, 26), lambda b, i: (b, i, 0)),
        out_shape=jax.ShapeDtypeStruct((B, N, 26), f32),
    )(pts, sampled, cur, m, W1.T, b1.reshape(1, -1), W2.T,
      b2.reshape(1, -1))
    return out


# speedup vs baseline: 1.7533x; 1.7533x over previous
import functools

import jax
import jax.numpy as jnp
from jax import lax
from jax.experimental import pallas as pl
from jax.experimental.pallas import tpu as pltpu
from jax.experimental.pallas import tpu_sc as plsc

IMAGE_FEAT_DIM = 64
FUSED_IMAGE_DIM = 16
TIME_COL = 6

_NC = 2
_NS = 16
_NW = _NC * _NS
_GB = 128


def _tmin_body(times_ref, tmin_ref):
    b = pl.program_id(0)
    tmin_ref[b, 0, 0] = jnp.min(times_ref[0, 0, :])


def _prep_body(pts_t_ref, tmin_ref, t_ref, p_ref, hw_ref,
               idx_ref, w_ref, cur_ref, m_ref, *, hf, wf):
    b = pl.program_id(0)
    x = pts_t_ref[0, 0, :]
    y = pts_t_ref[0, 1, :]
    z = pts_t_ref[0, 2, :]
    tm = pts_t_ref[0, TIME_COL, :]

    ct = tmin_ref[b, 0, 0]
    cur = jnp.abs(tm - ct) <= 1e-4

    def trow(i):
        return (x * t_ref[i, 0] + y * t_ref[i, 1] + z * t_ref[i, 2]
                + t_ref[i, 3])

    cp0 = trow(0)
    cp1 = trow(1)
    cp2 = trow(2)
    depth = cp2
    cp3 = x * t_ref[3, 0] + y * t_ref[3, 1] + z * t_ref[3, 2] + t_ref[3, 3]

    def prow(i):
        return (p_ref[i, 0] * cp0 + p_ref[i, 1] * cp1 + p_ref[i, 2] * cp2
                + p_ref[i, 3] * cp3)

    uvw0 = prow(0)
    uvw1 = prow(1)
    uvw2 = prow(2)
    den = jnp.maximum(uvw2, 1e-5)
    u = uvw0 / den
    v = uvw1 / den

    img_h = hw_ref[0]
    img_w = hw_ref[1]
    valid = ((depth > 0.0)
             & (u >= 0.0) & (u <= jnp.maximum(img_w - 1.0, 0.0))
             & (v >= 0.0) & (v <= jnp.maximum(img_h - 1.0, 0.0)))
    gx = 2.0 * (u / jnp.maximum(img_w - 1.0, 1.0)) - 1.0
    gy = 2.0 * (v / jnp.maximum(img_h - 1.0, 1.0)) - 1.0
    xs = (gx + 1.0) * 0.5 * (wf - 1)
    ys = (gy + 1.0) * 0.5 * (hf - 1)
    x0 = jnp.floor(xs)
    y0 = jnp.floor(ys)
    fx = xs - x0
    fy = ys - y0

    m = (valid & cur).astype(jnp.float32)
    taps = ((x0, y0, (1.0 - fx) * (1.0 - fy)),
            (x0 + 1.0, y0, fx * (1.0 - fy)),
            (x0, y0 + 1.0, (1.0 - fx) * fy),
            (x0 + 1.0, y0 + 1.0, fx * fy))
    for ti, (xf, yf, wgt) in enumerate(taps):
        inr = ((xf >= 0.0) & (xf <= wf - 1.0)
               & (yf >= 0.0) & (yf <= hf - 1.0))
        w_ref[0, ti, :] = wgt * inr.astype(jnp.float32) * m
    y0i = jnp.clip(y0, 0.0, hf - 1.0).astype(jnp.int32)
    x0i = jnp.clip(x0, 0.0, wf - 1.0).astype(jnp.int32)
    x1i = jnp.clip(x0 + 1.0, 0.0, wf - 1.0).astype(jnp.int32)
    rbase = (b * hf + y0i) * wf
    idx_ref[0, 0, :] = rbase + x0i
    idx_ref[0, 1, :] = rbase + x1i
    cur_ref[0, 0, :] = cur.astype(jnp.float32)
    m_ref[0, 0, :] = m


def _sc_gather_body(table_ref, idx_ref, w_ref, out_ref,
                    idx_v0, idx_v1, w_v0, w_v1, rows_v0, rows_v1,
                    out_v0, out_v1, sem_s0, sem_s1, sem_g0, sem_g1,
                    sem_o0, sem_o1, *, n_per_batch, nb):
    cid = lax.axis_index("c")
    sid = lax.axis_index("s")
    wid = sid * _NC + cid
    wpb = _NW // nb
    chunk = n_per_batch // wpb
    nblk = chunk // _GB
    bb = wid // wpb
    base = (wid % wpb) * chunk

    bufs = ((idx_v0, w_v0, rows_v0, out_v0, sem_s0, sem_g0, sem_o0),
            (idx_v1, w_v1, rows_v1, out_v1, sem_s1, sem_g1, sem_o1))

    def stage_start(j, k):
        idx_v, w_v, _, _, sem_s, _, _ = bufs[k]
        start = base + j * _GB
        for t in range(2):
            pltpu.async_copy(idx_ref.at[bb, t, pl.ds(start, _GB)],
                             idx_v.at[t], sem_s)
        for t in range(4):
            pltpu.async_copy(w_ref.at[bb, t, pl.ds(start, _GB)],
                             w_v.at[t], sem_s)

    def stage_wait(k):
        idx_v, w_v, _, _, sem_s, _, _ = bufs[k]
        for t in range(2):
            pltpu.make_async_copy(idx_ref.at[bb, t, pl.ds(base, _GB)],
                                  idx_v.at[t], sem_s).wait()
        for t in range(4):
            pltpu.make_async_copy(w_ref.at[bb, t, pl.ds(base, _GB)],
                                  w_v.at[t], sem_s).wait()

    def gather_start(k):
        idx_v, _, rows_v, _, _, sem_g, _ = bufs[k]
        for t in range(2):
            pltpu.async_copy(table_ref.at[idx_v.at[t]], rows_v.at[t], sem_g)

    def gather_wait(k):
        idx_v, _, rows_v, _, _, sem_g, _ = bufs[k]
        for t in range(2):
            pltpu.make_async_copy(table_ref.at[idx_v.at[t]], rows_v.at[t],
                                  sem_g).wait()

    def store_start(j, k):
        _, _, _, out_v, _, _, sem_o = bufs[k]
        start = base + j * _GB
        pltpu.async_copy(out_v, out_ref.at[bb, pl.ds(start, _GB)], sem_o)

    def store_wait(k):
        _, _, _, out_v, _, _, sem_o = bufs[k]
        pltpu.make_async_copy(out_v, out_ref.at[bb, pl.ds(base, _GB)],
                              sem_o).wait()

    def combine(k):
        _, w_v, rows_v, out_v, _, _, _ = bufs[k]

        def grp_body(g, c2):
            g16 = g * 16
            wvec = [w_v[t, pl.ds(g16, 16)] for t in range(4)]
            for kk in range(16):
                p = g16 + kk
                ws = [wvec[t][kk] for t in range(4)]
                for c in range(IMAGE_FEAT_DIM // 16):
                    c16 = c * 16
                    sl = pl.ds(c16, 16)
                    sh = pl.ds(IMAGE_FEAT_DIM + c16, 16)
                    acc = (rows_v[0, p, sl] * ws[0]
                           + rows_v[1, p, sl] * ws[1]
                           + rows_v[0, p, sh] * ws[2]
                           + rows_v[1, p, sh] * ws[3])
                    out_v[p, sl] = acc
            return c2

        lax.fori_loop(0, _GB // 16, grp_body, 0, unroll=False)

    stage_start(0, 0)
    stage_start(1, 1)
    stage_wait(0)
    gather_start(0)

    def body(i, carry):
        for k in range(2):
            j = 2 * i + k
            nk = 1 - k

            @pl.when(j + 1 < nblk)
            def _next():
                stage_wait(nk)
                gather_start(nk)

            gather_wait(k)

            @pl.when(j >= 2)
            def _drain():
                store_wait(k)

            combine(k)
            store_start(j, k)

            @pl.when(j + 2 < nblk)
            def _restage():
                stage_start(j + 2, k)
        return carry

    lax.fori_loop(0, nblk // 2, body, 0, unroll=False)
    store_wait(0)
    store_wait(1)


def _mlp_body(pts_ref, samp_ref, cur_ref, m_ref, w1t_ref, b1_ref, w2t_ref,
              b2_ref, out_ref):
    s = samp_ref[0]
    h = jnp.dot(s, w1t_ref[...], preferred_element_type=jnp.float32)
    h = jnp.maximum(h + b1_ref[...], 0.0)
    f = jnp.dot(h, w2t_ref[...], preferred_element_type=jnp.float32)
    mcol = m_ref[0, 0, :][:, None]
    f = (f + b2_ref[...]) * mcol
    out_ref[0, :, 0:8] = pts_ref[0]
    out_ref[0, :, 8:9] = cur_ref[0, 0, :][:, None]
    out_ref[0, :, 9:10] = mcol
    out_ref[0, :, 10:26] = f


def kernel(pts, img_feats, t_camera_radar, camera_projection, img_shape,
           W1, b1, W2, b2):
    B, N, D = pts.shape
    C, Hf, Wf = img_feats.shape[1], img_feats.shape[2], img_feats.shape[3]
    f32 = jnp.float32

    pts_t = jnp.transpose(pts, (0, 2, 1))
    times = pts_t[:, TIME_COL:TIME_COL + 1, :]
    table = jnp.transpose(img_feats, (0, 2, 3, 1)).reshape(B * Hf * Wf, C)
    south = jnp.concatenate([table[Wf:], jnp.zeros((Wf, C), f32)], axis=0)
    table = jnp.concatenate([table, south], axis=1)
    hw_f = img_shape.astype(f32)

    tmin = pl.pallas_call(
        _tmin_body,
        grid=(B,),
        in_specs=[pl.BlockSpec((1, 1, N), lambda b: (b, 0, 0))],
        out_specs=pl.BlockSpec(memory_space=pltpu.SMEM),
        out_shape=jax.ShapeDtypeStruct((B, 1, 1), f32),
    )(times)

    BLK = 2048
    NB = N // BLK
    smem = functools.partial(pl.BlockSpec, memory_space=pltpu.SMEM)
    idx4, w4, cur, m = pl.pallas_call(
        functools.partial(_prep_body, hf=Hf, wf=Wf),
        grid=(B, NB),
        in_specs=[
            pl.BlockSpec((1, 8, BLK), lambda b, i: (b, 0, i)),
            smem(),
            smem(),
            smem(),
            smem(),
        ],
        out_specs=[
            pl.BlockSpec((1, 2, BLK), lambda b, i: (b, 0, i)),
            pl.BlockSpec((1, 4, BLK), lambda b, i: (b, 0, i)),
            pl.BlockSpec((1, 1, BLK), lambda b, i: (b, 0, i)),
            pl.BlockSpec((1, 1, BLK), lambda b, i: (b, 0, i)),
        ],
        out_shape=[
            jax.ShapeDtypeStruct((B, 2, N), jnp.int32),
            jax.ShapeDtypeStruct((B, 4, N), f32),
            jax.ShapeDtypeStruct((B, 1, N), f32),
            jax.ShapeDtypeStruct((B, 1, N), f32),
        ],
    )(pts_t, tmin, t_camera_radar, camera_projection, hw_f)

    mesh = plsc.VectorSubcoreMesh(core_axis_name="c", subcore_axis_name="s",
                                  num_cores=_NC, num_subcores=_NS)
    sampled = pl.kernel(
        functools.partial(_sc_gather_body, n_per_batch=N, nb=B),
        out_type=jax.ShapeDtypeStruct((B, N, C), f32),
        mesh=mesh,
        scratch_types=[
            pltpu.VMEM((2, _GB), jnp.int32),
            pltpu.VMEM((2, _GB), jnp.int32),
            pltpu.VMEM((4, _GB), f32),
            pltpu.VMEM((4, _GB), f32),
            pltpu.VMEM((2, _GB, 2 * C), f32),
            pltpu.VMEM((2, _GB, 2 * C), f32),
            pltpu.VMEM((_GB, C), f32),
            pltpu.VMEM((_GB, C), f32),
            pltpu.SemaphoreType.DMA,
            pltpu.SemaphoreType.DMA,
            pltpu.SemaphoreType.DMA,
            pltpu.SemaphoreType.DMA,
            pltpu.SemaphoreType.DMA,
            pltpu.SemaphoreType.DMA,
        ],
    )(table, idx4, w4)

    BLK2 = 2048
    out = pl.pallas_call(
        _mlp_body,
        grid=(B, N // BLK2),
        in_specs=[
            pl.BlockSpec((1, BLK2, 8), lambda b, i: (b, i, 0)),
            pl.BlockSpec((1, BLK2, C), lambda b, i: (b, i, 0)),
            pl.BlockSpec((1, 1, BLK2), lambda b, i: (b, 0, i)),
            pl.BlockSpec((1, 1, BLK2), lambda b, i: (b, 0, i)),
            pl.BlockSpec((C, FUSED_IMAGE_DIM), lambda b, i: (0, 0)),
            pl.BlockSpec((1, FUSED_IMAGE_DIM), lambda b, i: (0, 0)),
            pl.BlockSpec((FUSED_IMAGE_DIM, FUSED_IMAGE_DIM),
                         lambda b, i: (0, 0)),
            pl.BlockSpec((1, FUSED_IMAGE_DIM), lambda b, i: (0, 0)),
        ],
        out_specs=pl.BlockSpec((1, BLK2, 26), lambda b, i: (b, i, 0)),
        out_shape=jax.ShapeDtypeStruct((B, N, 26), f32),
    )(pts, sampled, cur, m, W1.T, b1.reshape(1, -1), W2.T,
      b2.reshape(1, -1))
    return out
